# Initial kernel scaffold; baseline (speedup 1.0000x reference)
import jax
import jax.numpy as jnp
from jax.experimental import pallas as pl


def kernel(indices, weight):
    raise NotImplementedError("write your pallas kernel here")


# speedup vs baseline: 1.4409x; 1.4409x over previous
import functools

import jax
import jax.numpy as jnp
from jax import lax
from jax.experimental import pallas as pl
from jax.experimental.pallas import tpu as pltpu
from jax.experimental.pallas import tpu_sc as plsc

CHUNK = 128


def kernel(indices, weight):
    B, F = indices.shape
    V, D = weight.shape
    N = B * F

    info = plsc.get_sparse_core_info()
    NC, NS = info.num_cores, info.num_subcores
    NW = NC * NS
    per_w = N // NW
    n_chunks = per_w // CHUNK
    assert per_w * NW == N and n_chunks * CHUNK == per_w

    idx = indices.reshape(NW, n_chunks, CHUNK).astype(jnp.int32)
    mesh = plsc.VectorSubcoreMesh(core_axis_name="c", subcore_axis_name="s")

    @functools.partial(
        pl.kernel,
        out_type=jax.ShapeDtypeStruct((NW, n_chunks, CHUNK, D), jnp.float32),
        mesh=mesh,
        scratch_types=[
            pltpu.VMEM((n_chunks, CHUNK), jnp.int32),
            pltpu.VMEM((CHUNK, D), jnp.float32),
            pltpu.SemaphoreType.DMA,
        ],
        compiler_params=pltpu.CompilerParams(use_tc_tiling_on_sc=False),
    )
    def emb(idx_hbm, table_hbm, out_hbm, idx_v, rows_v, gsem):
        wid = lax.axis_index("s") * NC + lax.axis_index("c")
        pltpu.sync_copy(idx_hbm.at[wid], idx_v)

        @pl.loop(0, n_chunks)
        def _(j):
            pltpu.async_copy(table_hbm.at[idx_v.at[j]], rows_v, gsem).wait()
            pltpu.sync_copy(rows_v, out_hbm.at[wid, j])

    out = emb(idx, weight)
    return out.reshape(B, F, D)


# speedup vs baseline: 1.5676x; 1.0879x over previous
import functools

import jax
import jax.numpy as jnp
from jax import lax
from jax.experimental import pallas as pl
from jax.experimental.pallas import tpu as pltpu
from jax.experimental.pallas import tpu_sc as plsc

CHUNK = 128


def kernel(indices, weight):
    B, F = indices.shape
    V, D = weight.shape
    N = B * F

    info = plsc.get_sparse_core_info()
    NC, NS = info.num_cores, info.num_subcores
    NW = NC * NS
    per_w = N // NW
    n_chunks = per_w // CHUNK
    assert per_w * NW == N and n_chunks * CHUNK == per_w

    K = 13
    G = n_chunks // K
    assert G * K == n_chunks and G % 2 == 0
    GROUP = K * CHUNK

    idx = indices.reshape(NW, G, K, CHUNK).astype(jnp.int32)
    mesh = plsc.VectorSubcoreMesh(core_axis_name="c", subcore_axis_name="s")

    @functools.partial(
        pl.kernel,
        out_type=jax.ShapeDtypeStruct((NW, G, GROUP, D), jnp.float32),
        mesh=mesh,
        scratch_types=[
            pltpu.VMEM((G, K, CHUNK), jnp.int32),
            pltpu.VMEM((2, GROUP, D), jnp.float32),
            pltpu.SemaphoreType.DMA,
            pltpu.SemaphoreType.DMA,
            pltpu.SemaphoreType.DMA,
        ],
        compiler_params=pltpu.CompilerParams(use_tc_tiling_on_sc=False),
    )
    def emb(idx_hbm, table_hbm, out_hbm, idx_v, rows_v, gsem, ssem0, ssem1):
        wid = lax.axis_index("s") * NC + lax.axis_index("c")
        pltpu.sync_copy(idx_hbm.at[wid], idx_v)
        ssems = (ssem0, ssem1)

        def gather_group(g, b):
            descs = [
                pltpu.async_copy(
                    table_hbm.at[idx_v.at[g, k]],
                    rows_v.at[b, pl.ds(k * CHUNK, CHUNK)],
                    gsem,
                )
                for k in range(K)
            ]
            for d_ in descs:
                d_.wait()

        def fire_store(g, b):
            pltpu.async_copy(rows_v.at[b], out_hbm.at[wid, g], ssems[b])

        def wait_store(b):
            pltpu.make_async_copy(rows_v.at[b], out_hbm.at[wid, 0], ssems[b]).wait()

        gather_group(0, 0)
        fire_store(0, 0)
        gather_group(1, 1)
        fire_store(1, 1)

        @pl.loop(2, G, step=2)
        def _(g):
            for b in range(2):
                wait_store(b)
                gather_group(g + b, b)
                fire_store(g + b, b)

        wait_store(0)
        wait_store(1)

    out = emb(idx, weight)
    return out.reshape(B, F, D)
